# baseline (speedup 1.0000x reference)
import functools

import jax
import jax.numpy as jnp
from jax import lax
from jax.experimental import pallas as pl
from jax.experimental.pallas import tpu as pltpu
from jax.experimental.pallas import tpu_sc as plsc

_ROWS = 16384
_COLS = 64
_NC = 2
_NS = 16
_SHARD = _ROWS // (_NC * _NS)

_mesh = plsc.VectorSubcoreMesh(core_axis_name="c", subcore_axis_name="s")


@functools.partial(
    pl.kernel,
    mesh=_mesh,
    out_type=jax.ShapeDtypeStruct((_ROWS, _COLS), jnp.float32),
    scratch_types=[
        pltpu.VMEM((_SHARD, _COLS), jnp.float32),
        pltpu.SemaphoreType.DMA,
    ],
)
def _sc_copy(x_hbm, out_hbm, buf, sem):
    wid = lax.axis_index("s") * _NC + lax.axis_index("c")
    base = wid * _SHARD
    pltpu.async_copy(x_hbm.at[pl.ds(base, _SHARD), :], buf, sem).wait()
    pltpu.async_copy(buf, out_hbm.at[pl.ds(base, _SHARD), :], sem).wait()


def kernel(x):
    return _sc_copy(x)


# speedup vs baseline: 1.7993x; 1.7993x over previous
import jax
from jax.experimental import pallas as pl
from jax.experimental.pallas import tpu as pltpu

_N_CHUNKS = 16


def _copy_body(x_ref, o_ref, buf, in_sems, out_sems):
    rows = x_ref.shape[0]
    chunk = rows // _N_CHUNKS
    for i in range(_N_CHUNKS):
        sl = pl.ds(i * chunk, chunk)
        pltpu.make_async_copy(x_ref.at[sl, :], buf.at[sl, :], in_sems.at[i]).start()
    for i in range(_N_CHUNKS):
        sl = pl.ds(i * chunk, chunk)
        pltpu.make_async_copy(x_ref.at[sl, :], buf.at[sl, :], in_sems.at[i]).wait()
        pltpu.make_async_copy(buf.at[sl, :], o_ref.at[sl, :], out_sems.at[i]).start()
    for i in range(_N_CHUNKS):
        sl = pl.ds(i * chunk, chunk)
        pltpu.make_async_copy(buf.at[sl, :], o_ref.at[sl, :], out_sems.at[i]).wait()


def kernel(x):
    return pl.pallas_call(
        _copy_body,
        in_specs=[pl.BlockSpec(memory_space=pl.ANY)],
        out_specs=pl.BlockSpec(memory_space=pl.ANY),
        out_shape=jax.ShapeDtypeStruct(x.shape, x.dtype),
        scratch_shapes=[
            pltpu.VMEM(x.shape, x.dtype),
            pltpu.SemaphoreType.DMA((_N_CHUNKS,)),
            pltpu.SemaphoreType.DMA((_N_CHUNKS,)),
        ],
    )(x)
